# baseline (speedup 1.0000x reference)
import functools

import jax
import jax.numpy as jnp
from jax import lax
from jax.experimental import pallas as pl
from jax.experimental.pallas import tpu as pltpu
from jax.experimental.pallas import tpu_sc as plsc

N = 100000
X = 128
H = 128
G = 256
NCLS = 104

NC = 2
NS = 16
NW = NC * NS

N_PAD = 102400
E = N - 1
C_EDGE = 3328
NWIN = C_EDGE // 256
K_WIN = 256
E_PAD = NW * C_EDGE
LA = 16 + E_PAD + 16
OUTB = 256
NREC = 2 * NW

_mesh = plsc.VectorSubcoreMesh(
    core_axis_name="c", subcore_axis_name="s", num_cores=NC, num_subcores=NS)
_sc_params = pltpu.CompilerParams(needs_layout_passes=False)


def _wid():
  return lax.axis_index("s") * NC + lax.axis_index("c")


@functools.partial(
    pl.kernel,
    out_type=jax.ShapeDtypeStruct((N_PAD, X), jnp.float32),
    mesh=_mesh,
    compiler_params=_sc_params,
    scratch_types=[
        pltpu.VMEM((320,), jnp.int32),
        pltpu.VMEM((320, X), jnp.float32),
        pltpu.SemaphoreType.DMA,
    ],
)
def _emb_gather(nt_hbm, emb_hbm, h0_hbm, idx_v, rows_v, sem):
  w = _wid()
  base0 = w * (N_PAD // NW)

  def step(i, _):
    base = pl.multiple_of(base0 + i * 320, 64)
    pltpu.sync_copy(nt_hbm.at[pl.ds(base, 320)], idx_v)
    pltpu.async_copy(emb_hbm.at[idx_v], rows_v, sem).wait()
    pltpu.sync_copy(rows_v, h0_hbm.at[pl.ds(base, 320)])
    return 0

  lax.fori_loop(0, N_PAD // NW // 320, step, 0)


@functools.partial(
    pl.kernel,
    out_type=(
        jax.ShapeDtypeStruct((N_PAD, X), jnp.float32),
        jax.ShapeDtypeStruct((N_PAD, X), jnp.float32),
        jax.ShapeDtypeStruct((NW * 512,), jnp.float32),
        jax.ShapeDtypeStruct((NW * 16,), jnp.float32),
    ),
    mesh=_mesh,
    compiler_params=_sc_params,
    scratch_types=[
        pltpu.VMEM((288,), jnp.int32),
        pltpu.VMEM((K_WIN,), jnp.int32),
        pltpu.VMEM((K_WIN, X), jnp.float32),
        pltpu.VMEM((OUTB, X), jnp.float32),
        pltpu.VMEM((OUTB, X), jnp.float32),
        pltpu.VMEM((OUTB,), jnp.int32),
        pltpu.VMEM((OUTB,), jnp.float32),
        pltpu.VMEM((512,), jnp.float32),
        pltpu.VMEM((16,), jnp.float32),
        pltpu.SemaphoreType.DMA,
    ],
)
def _edge_pass(src_hbm, dst_hbm, h0_hbm, t0_hbm, t1_hbm, part_hbm, meta_hbm,
               dstw, srcw, rows, out_t0, out_t1, out_dst, out_cnt,
               pstage, mstage, sem):
  w = _wid()
  t0 = w * C_EDGE
  dump = N + 16 + w
  lanes = lax.iota(jnp.int32, 16)
  lane0 = lanes == 0
  zero16 = jnp.zeros((16,), jnp.float32)
  cols = [16 * c + lanes for c in range(8)]

  def reset_outdst():
    dv = jnp.full((16,), dump, jnp.int32)
    for j in range(OUTB // 16):
      out_dst[pl.ds(16 * j, 16)] = dv

  def drain():
    pltpu.async_copy(out_t0, t0_hbm.at[out_dst], sem).wait()
    pltpu.async_copy(out_t1, t1_hbm.at[out_dst], sem).wait()

  def splat_i(x):
    return jnp.full((16,), x, jnp.int32)

  mstage[...] = jnp.where((lanes == 0) | (lanes == 8), -1.0, 0.0)

  def stage(i):
    base = pl.multiple_of(t0 + i * K_WIN, 8)
    pltpu.sync_copy(dst_hbm.at[pl.ds(base, 288)], dstw)
    pltpu.sync_copy(src_hbm.at[pl.ds(base + 16, K_WIN)], srcw)
    pltpu.async_copy(h0_hbm.at[srcw], rows, sem).wait()

  stage(0)
  curv = plsc.load_gather(dstw, [splat_i(15)])
  cur0 = curv[0]
  cont = plsc.load_gather(dstw, [splat_i(16)])[0] == cur0
  first_done = jnp.where(cont, jnp.int32(0), jnp.int32(1))
  posv = zero16
  a0 = [zero16] * 8
  a1 = [zero16] * 8

  for i in range(NWIN):
    if i > 0:
      stage(i)
    reset_outdst()

    def edge(j, c):
      dvidx, curv, posv, noutv = c[:4]
      a0 = list(c[4:12])
      a1 = list(c[12:20])
      dv = plsc.load_gather(dstw, [dvidx])
      changed = dv != curv
      flush = changed & (posv > 0.0)
      flushd = flush & lane0
      for cc in range(8):
        plsc.store_scatter(out_t0, [noutv, cols[cc]], a0[cc], mask=flush)
        plsc.store_scatter(out_t1, [noutv, cols[cc]], a1[cc], mask=flush)
      plsc.store_scatter(out_dst, [noutv], curv, mask=flushd)
      plsc.store_scatter(out_cnt, [noutv], posv, mask=flushd)
      noutv2 = noutv + jnp.where(flush, 1, 0).astype(jnp.int32)
      posb = jnp.where(changed, 0.0, posv)
      jv = dvidx - 16
      na0, na1 = [], []
      for cc in range(8):
        r = plsc.load_gather(rows, [jv, cols[cc]])
        na0.append(jnp.where(changed, zero16, a0[cc]) + r)
        na1.append(jnp.where(changed, zero16, a1[cc]) + posb * r)
      return tuple([dvidx + 1, dv, posb + 1.0, noutv2] + na0 + na1)

    init = tuple([splat_i(16), curv, posv, splat_i(0)] + a0 + a1)
    fin = lax.fori_loop(0, K_WIN, edge, init)
    curv, posv = fin[1], fin[2]
    noutv = fin[3]
    a0 = list(fin[4:12])
    a1 = list(fin[12:20])

    nflushed = noutv[0]
    do_fix = cont & (first_done == 0) & (nflushed > 0)

    @pl.when(do_fix)
    def _():
      for cc in range(8):
        pstage[pl.ds(16 * cc, 16)] = out_t0[0, pl.ds(16 * cc, 16)]
        pstage[pl.ds(128 + 16 * cc, 16)] = out_t1[0, pl.ds(16 * cc, 16)]
      cnt0 = plsc.load_gather(out_cnt, [splat_i(0)])
      mv = mstage[...]
      mv = jnp.where(lanes == 0, cur0.astype(jnp.float32), mv)
      mv = jnp.where(lanes == 1, cnt0, mv)
      mstage[...] = mv
      dv0 = out_dst[pl.ds(0, 16)]
      out_dst[pl.ds(0, 16)] = jnp.where(lane0, dump, dv0)

    first_done = jnp.where(do_fix, jnp.int32(1), first_done)
    drain()

  nxt = plsc.load_gather(dstw, [splat_i(272)])[0]
  cur = curv[0]
  ends_after = nxt == cur
  is_first = cont & (first_done == 0)

  @pl.when(is_first)
  def _():
    for cc in range(8):
      pstage[pl.ds(16 * cc, 16)] = a0[cc]
      pstage[pl.ds(128 + 16 * cc, 16)] = a1[cc]
    mv = mstage[...]
    mv = jnp.where(lanes == 0, curv.astype(jnp.float32), mv)
    mv = jnp.where(lanes == 1, posv, mv)
    mstage[...] = mv

  @pl.when(jnp.logical_not(is_first) & ends_after)
  def _():
    for cc in range(8):
      pstage[pl.ds(256 + 16 * cc, 16)] = a0[cc]
      pstage[pl.ds(384 + 16 * cc, 16)] = a1[cc]
    mv = mstage[...]
    mv = jnp.where(lanes == 8, curv.astype(jnp.float32), mv)
    mv = jnp.where(lanes == 9, posv, mv)
    mstage[...] = mv

  @pl.when(jnp.logical_not(is_first) & jnp.logical_not(ends_after))
  def _():
    reset_outdst()
    for cc in range(8):
      out_t0[0, pl.ds(16 * cc, 16)] = a0[cc]
      out_t1[0, pl.ds(16 * cc, 16)] = a1[cc]
    dv0 = out_dst[pl.ds(0, 16)]
    out_dst[pl.ds(0, 16)] = jnp.where(lane0, curv, dv0)
    drain()

  pltpu.sync_copy(pstage, part_hbm.at[pl.ds(w * 512, 512)])
  pltpu.sync_copy(mstage, meta_hbm.at[pl.ds(w * 16, 16)])


@functools.partial(
    pl.kernel,
    out_type=(
        jax.ShapeDtypeStruct((NREC, X), jnp.float32),
        jax.ShapeDtypeStruct((NREC, X), jnp.float32),
        jax.ShapeDtypeStruct((NREC,), jnp.int32),
    ),
    mesh=_mesh,
    compiler_params=_sc_params,
    scratch_types=[
        pltpu.VMEM((NW * 512,), jnp.float32),
        pltpu.VMEM((NW * 16,), jnp.float32),
        pltpu.VMEM((NREC, X), jnp.float32),
        pltpu.VMEM((NREC, X), jnp.float32),
        pltpu.VMEM((NREC,), jnp.int32),
        pltpu.SemaphoreType.DMA,
    ],
)
def _merge(part_hbm, meta_hbm, cr_hbm, cl_hbm, bx_hbm,
           pv, mv, cr_v, cl_v, bx_v, sem):
  w = _wid()
  lanes = lax.iota(jnp.int32, 16)
  zero16 = jnp.zeros((16,), jnp.float32)

  pltpu.sync_copy(part_hbm, pv)
  pltpu.sync_copy(meta_hbm, mv)
  neg1 = jnp.full((16,), -1, jnp.int32)
  for r in range(NREC // 16):
    bx_v[pl.ds(16 * r, 16)] = neg1
  for r in range(NREC):
    for j8 in range(8):
      cr_v[r, pl.ds(16 * j8, 16)] = zero16
      cl_v[r, pl.ds(16 * j8, 16)] = zero16

  def rec(r, c):
    (have, cur, cnt, nfin) = c[:4]
    a0 = list(c[4:12])
    a1 = list(c[12:20])
    tile = r // 2
    slot = r - 2 * tile
    mb = tile * 16 + slot * 8
    pdstf = mv[pl.ds(mb, 16)][0]
    pcnt = mv[pl.ds(mb, 16)][1]
    pdst = pdstf.astype(jnp.int32)
    present = pdst >= 0
    same = present & (have == 1) & (pdst == cur)
    newopen = present & jnp.logical_not(same)
    fin_now = newopen & (have == 1)

    @pl.when(fin_now)
    def _():
      ridx = jnp.full((16,), nfin, jnp.int32)
      for j8 in range(8):
        cols = 16 * j8 + lanes
        plsc.store_scatter(cr_v, [ridx, cols], a0[j8])
        plsc.store_scatter(cl_v, [ridx, cols], a1[j8])
      plsc.store_scatter(bx_v, [ridx],
                         jnp.full((16,), cur, jnp.int32), mask=lanes == 0)

    nfin2 = nfin + fin_now.astype(jnp.int32)
    pb = tile * 512 + slot * 256
    na0, na1 = [], []
    for j8 in range(8):
      t0v = pv[pl.ds(pb + 16 * j8, 16)]
      t1v = pv[pl.ds(pb + 128 + 16 * j8, 16)]
      n0 = jnp.where(same, a0[j8] + t0v, jnp.where(newopen, t0v, a0[j8]))
      n1 = jnp.where(same, a1[j8] + t1v + cnt * t0v,
                     jnp.where(newopen, t1v, a1[j8]))
      na0.append(n0)
      na1.append(n1)
    cnt2 = jnp.where(same, cnt + pcnt, jnp.where(newopen, pcnt, cnt))
    cur2 = jnp.where(newopen, pdst, cur)
    have2 = jnp.where(newopen, 1, have)
    return tuple([have2, cur2, cnt2, nfin2] + na0 + na1)

  init = tuple([jnp.int32(0), jnp.int32(-1), jnp.float32(0.0), jnp.int32(0)]
               + [zero16] * 16)
  fin = lax.fori_loop(0, NREC, rec, init)
  (have, cur, cnt, nfin) = fin[:4]

  @pl.when(have == 1)
  def _():
    fa0, fa1 = list(fin[4:12]), list(fin[12:20])
    ridx = jnp.full((16,), nfin, jnp.int32)
    for j8 in range(8):
      cols = 16 * j8 + lanes
      plsc.store_scatter(cr_v, [ridx, cols], fa0[j8])
      plsc.store_scatter(cl_v, [ridx, cols], fa1[j8])
    plsc.store_scatter(bx_v, [ridx],
                       jnp.full((16,), cur, jnp.int32), mask=lanes == 0)

  @pl.when(w == 0)
  def _():
    pltpu.sync_copy(cr_v, cr_hbm)
    pltpu.sync_copy(cl_v, cl_hbm)
    pltpu.sync_copy(bx_v, bx_hbm)


BM = 512


def _conv_body(t0, t1, h0, deg, bidx, c0, c1, wr, wl, wt, b, h_out):
  i = pl.program_id(0)
  rel = bidx[...] - i * BM
  sel = (lax.broadcasted_iota(jnp.int32, (BM, NREC), 0) == rel)
  self_f = sel.astype(jnp.float32)
  hit = jnp.sum(self_f, axis=1, keepdims=True)
  t0_e = jnp.where(hit > 0.0,
                   jnp.dot(self_f, c0[...], preferred_element_type=jnp.float32),
                   t0[...])
  t1_e = jnp.where(hit > 0.0,
                   jnp.dot(self_f, c1[...], preferred_element_type=jnp.float32),
                   t1[...])
  degv = deg[...]
  inv = 1.0 / jnp.maximum(degv - 1.0, 1.0)
  sr_e = jnp.where(degv == 1.0, 0.5 * t0_e, t1_e * inv)
  sl_e = t0_e - sr_e
  cs = jnp.dot(sr_e, wr[...], preferred_element_type=jnp.float32)
  cs += jnp.dot(sl_e, wl[...], preferred_element_type=jnp.float32)
  cs += jnp.dot(h0[...], wt[...], preferred_element_type=jnp.float32)
  hn = jnp.maximum(cs + b[...], 0.0)
  h_out[...] = jnp.where(deg[...] > 0.0, hn, h0[...])


def _conv(sr, sl, h0, deg, bidx, cr, cl, wr, wl, wt, b):
  nb = N_PAD // BM
  blk = lambda i: (i, 0)
  cst = lambda i: (0, 0)
  return pl.pallas_call(
      _conv_body,
      grid=(nb,),
      in_specs=[
          pl.BlockSpec((BM, X), blk),
          pl.BlockSpec((BM, X), blk),
          pl.BlockSpec((BM, X), blk),
          pl.BlockSpec((BM, 1), blk),
          pl.BlockSpec((1, NREC), cst),
          pl.BlockSpec((NREC, X), cst),
          pl.BlockSpec((NREC, X), cst),
          pl.BlockSpec((X, H), cst),
          pl.BlockSpec((X, H), cst),
          pl.BlockSpec((X, H), cst),
          pl.BlockSpec((1, H), cst),
      ],
      out_specs=pl.BlockSpec((BM, H), blk),
      out_shape=jax.ShapeDtypeStruct((N_PAD, H), jnp.float32),
  )(sr, sl, h0, deg, bidx, cr, cl, wr, wl, wt, b)


NEG = -1e30


def _pool_body(h, gid, gw, gb, cw, cb, out, m_s, z_s, p_s):
  i = pl.program_id(0)
  nb = pl.num_programs(0)

  @pl.when(i == 0)
  def _():
    m_s[...] = jnp.full((G, 1), NEG, jnp.float32)
    z_s[...] = jnp.zeros((G, 1), jnp.float32)
    p_s[...] = jnp.zeros((G, H), jnp.float32)

  hb = h[...]
  s = lax.dot_general(gw[...], hb, (((1,), (1,)), ((), ()))) + gb[0, 0]
  gids = gid[0]
  mask = lax.broadcasted_iota(jnp.int32, (G, BM), 0) == gids
  bmax = jnp.max(jnp.where(mask, s, NEG), axis=1, keepdims=True)
  m_old = m_s[...]
  m_new = jnp.maximum(m_old, bmax)
  corr = jnp.where(m_new == m_old, 1.0, jnp.exp(m_old - m_new))
  m_s[...] = m_new
  e = jnp.where(mask, jnp.exp(s - m_new), 0.0)
  z_s[...] = z_s[...] * corr + jnp.sum(e, axis=1, keepdims=True)
  p_s[...] = p_s[...] * corr + jnp.dot(e, hb, preferred_element_type=jnp.float32)

  @pl.when(i == nb - 1)
  def _():
    pooled = p_s[...] / jnp.maximum(z_s[...], 1e-30)
    out[...] = lax.dot_general(
        pooled, cw[...], (((1,), (1,)), ((), ()))) + cb[...]


def _pool(h, gid, gw, gb, cw, cb):
  nb = N_PAD // BM
  blk = lambda i: (i, 0)
  cst = lambda i: (0, 0)
  return pl.pallas_call(
      _pool_body,
      grid=(nb,),
      in_specs=[
          pl.BlockSpec((BM, H), blk),
          pl.BlockSpec((1, 1, BM), lambda i: (i, 0, 0)),
          pl.BlockSpec((1, H), cst),
          pl.BlockSpec((1, 1), cst),
          pl.BlockSpec((NCLS, H), cst),
          pl.BlockSpec((1, NCLS), cst),
      ],
      out_specs=pl.BlockSpec((G, NCLS), cst),
      out_shape=jax.ShapeDtypeStruct((G, NCLS), jnp.float32),
      scratch_shapes=[
          pltpu.VMEM((G, 1), jnp.float32),
          pltpu.VMEM((G, 1), jnp.float32),
          pltpu.VMEM((G, H), jnp.float32),
      ],
  )(h, gid, gw, gb, cw, cb)


def kernel(node_types, edge_index, graph_ids, emb, W_left, W_right, W_top,
           b_conv, gate_W, gate_b, cls_W, cls_b):
  nt = node_types.astype(jnp.int32)
  src = edge_index[0].astype(jnp.int32)
  dst = edge_index[1].astype(jnp.int32)
  gid = graph_ids.astype(jnp.int32)

  nt_pad = jnp.concatenate([nt, jnp.zeros((N_PAD - N,), jnp.int32)])
  src_arr = jnp.concatenate([
      jnp.zeros((16,), jnp.int32), src, jnp.zeros((LA - 16 - E,), jnp.int32)])
  dst_arr = jnp.concatenate([
      jnp.full((16,), -1, jnp.int32), dst, jnp.full((LA - 16 - E,), N,
                                                    jnp.int32)])
  gid_pad = jnp.concatenate([gid, jnp.full((N_PAD - N,), G, jnp.int32)])

  deg = jax.ops.segment_sum(jnp.ones((E,), jnp.float32), dst, num_segments=N,
                            indices_are_sorted=True)
  deg_pad = jnp.concatenate([deg, jnp.zeros((N_PAD - N,), jnp.float32)])

  h0 = _emb_gather(nt_pad, emb)
  s_r, s_l, part, meta = _edge_pass(src_arr, dst_arr, h0)
  c_r, c_l, bidx = _merge(part, meta)
  h = _conv(s_r, s_l, h0, deg_pad.reshape(N_PAD, 1), bidx.reshape(1, NREC),
            c_r, c_l, W_right, W_left, W_top, b_conv)
  logits = _pool(h, gid_pad.reshape(N_PAD // BM, 1, BM), gate_W,
                 gate_b.reshape(1, 1), cls_W, cls_b.reshape(1, NCLS))
  return logits


# speedup vs baseline: 1.0745x; 1.0745x over previous
import functools

import jax
import jax.numpy as jnp
from jax import lax
from jax.experimental import pallas as pl
from jax.experimental.pallas import tpu as pltpu
from jax.experimental.pallas import tpu_sc as plsc

N = 100000
X = 128
H = 128
G = 256
NCLS = 104

NC = 2
NS = 16
NW = NC * NS

N_PAD = 102400
E = N - 1
C_EDGE = 3328
NWIN = C_EDGE // 256
K_WIN = 256
E_PAD = NW * C_EDGE
LA = 16 + E_PAD + 16
OUTB = 256
NREC = 2 * NW

_mesh = plsc.VectorSubcoreMesh(
    core_axis_name="c", subcore_axis_name="s", num_cores=NC, num_subcores=NS)
_sc_params = pltpu.CompilerParams(needs_layout_passes=False)


def _wid():
  return lax.axis_index("s") * NC + lax.axis_index("c")


@functools.partial(
    pl.kernel,
    out_type=jax.ShapeDtypeStruct((N_PAD, X), jnp.float32),
    mesh=_mesh,
    compiler_params=_sc_params,
    scratch_types=[
        pltpu.VMEM((320,), jnp.int32),
        pltpu.VMEM((320, X), jnp.float32),
        pltpu.SemaphoreType.DMA,
    ],
)
def _emb_gather(nt_hbm, emb_hbm, h0_hbm, idx_v, rows_v, sem):
  w = _wid()
  base0 = w * (N_PAD // NW)

  def step(i, _):
    base = pl.multiple_of(base0 + i * 320, 64)
    pltpu.sync_copy(nt_hbm.at[pl.ds(base, 320)], idx_v)
    pltpu.async_copy(emb_hbm.at[idx_v], rows_v, sem).wait()
    pltpu.sync_copy(rows_v, h0_hbm.at[pl.ds(base, 320)])
    return 0

  lax.fori_loop(0, N_PAD // NW // 320, step, 0)


@functools.partial(
    pl.kernel,
    out_type=(
        jax.ShapeDtypeStruct((N_PAD, X), jnp.float32),
        jax.ShapeDtypeStruct((N_PAD, X), jnp.float32),
        jax.ShapeDtypeStruct((NW * 512,), jnp.float32),
        jax.ShapeDtypeStruct((NW * 16,), jnp.float32),
    ),
    mesh=_mesh,
    compiler_params=_sc_params,
    scratch_types=[
        pltpu.VMEM((288,), jnp.int32),
        pltpu.VMEM((K_WIN,), jnp.int32),
        pltpu.VMEM((K_WIN, X), jnp.float32),
        pltpu.VMEM((OUTB, X), jnp.float32),
        pltpu.VMEM((OUTB, X), jnp.float32),
        pltpu.VMEM((OUTB,), jnp.int32),
        pltpu.VMEM((OUTB,), jnp.float32),
        pltpu.VMEM((512,), jnp.float32),
        pltpu.VMEM((16,), jnp.float32),
        pltpu.SemaphoreType.DMA,
    ],
)
def _edge_pass(src_hbm, dst_hbm, h0_hbm, t0_hbm, t1_hbm, part_hbm, meta_hbm,
               dstw, srcw, rows, out_t0, out_t1, out_dst, out_cnt,
               pstage, mstage, sem):
  w = _wid()
  t0 = w * C_EDGE
  dump = N + 16
  lanes = lax.iota(jnp.int32, 16)
  lane0 = lanes == 0
  zero16 = jnp.zeros((16,), jnp.float32)
  cols = [16 * c + lanes for c in range(8)]

  def reset_outdst():
    for j in range(OUTB // 16):
      out_dst[pl.ds(16 * j, 16)] = dump + 16 * j + lanes

  def drain():
    pltpu.async_copy(out_t0, t0_hbm.at[out_dst], sem).wait()
    pltpu.async_copy(out_t1, t1_hbm.at[out_dst], sem).wait()

  def splat_i(x):
    return jnp.full((16,), x, jnp.int32)

  mstage[...] = jnp.where((lanes == 0) | (lanes == 8), -1.0, 0.0)

  def stage(i):
    base = pl.multiple_of(t0 + i * K_WIN, 8)
    pltpu.sync_copy(dst_hbm.at[pl.ds(base, 288)], dstw)
    pltpu.sync_copy(src_hbm.at[pl.ds(base + 16, K_WIN)], srcw)
    pltpu.async_copy(h0_hbm.at[srcw], rows, sem).wait()

  stage(0)
  curv = plsc.load_gather(dstw, [splat_i(15)])
  cur0 = curv[0]
  cont = plsc.load_gather(dstw, [splat_i(16)])[0] == cur0
  first_done = jnp.where(cont, jnp.int32(0), jnp.int32(1))
  posv = zero16
  a0 = [zero16] * 8
  a1 = [zero16] * 8

  for i in range(NWIN):
    if i > 0:
      stage(i)
    reset_outdst()

    def edge(j, c):
      dvidx, curv, posv, noutv = c[:4]
      a0 = list(c[4:12])
      a1 = list(c[12:20])
      dv = plsc.load_gather(dstw, [dvidx])
      changed = dv != curv
      flush = changed & (posv > 0.0)
      flushd = flush & lane0
      for cc in range(8):
        plsc.store_scatter(out_t0, [noutv, cols[cc]], a0[cc], mask=flush)
        plsc.store_scatter(out_t1, [noutv, cols[cc]], a1[cc], mask=flush)
      plsc.store_scatter(out_dst, [noutv], curv, mask=flushd)
      plsc.store_scatter(out_cnt, [noutv], posv, mask=flushd)
      noutv2 = noutv + jnp.where(flush, 1, 0).astype(jnp.int32)
      posb = jnp.where(changed, 0.0, posv)
      jv = dvidx - 16
      na0, na1 = [], []
      for cc in range(8):
        r = plsc.load_gather(rows, [jv, cols[cc]])
        na0.append(jnp.where(changed, zero16, a0[cc]) + r)
        na1.append(jnp.where(changed, zero16, a1[cc]) + posb * r)
      return tuple([dvidx + 1, dv, posb + 1.0, noutv2] + na0 + na1)

    init = tuple([splat_i(16), curv, posv, splat_i(0)] + a0 + a1)
    fin = lax.fori_loop(0, K_WIN, edge, init)
    curv, posv = fin[1], fin[2]
    noutv = fin[3]
    a0 = list(fin[4:12])
    a1 = list(fin[12:20])

    nflushed = noutv[0]
    do_fix = cont & (first_done == 0) & (nflushed > 0)

    @pl.when(do_fix)
    def _():
      for cc in range(8):
        pstage[pl.ds(16 * cc, 16)] = out_t0[0, pl.ds(16 * cc, 16)]
        pstage[pl.ds(128 + 16 * cc, 16)] = out_t1[0, pl.ds(16 * cc, 16)]
      cnt0 = plsc.load_gather(out_cnt, [splat_i(0)])
      mv = mstage[...]
      mv = jnp.where(lanes == 0, cur0.astype(jnp.float32), mv)
      mv = jnp.where(lanes == 1, cnt0, mv)
      mstage[...] = mv
      dv0 = out_dst[pl.ds(0, 16)]
      out_dst[pl.ds(0, 16)] = jnp.where(lane0, dump, dv0)

    first_done = jnp.where(do_fix, jnp.int32(1), first_done)
    drain()

  nxt = plsc.load_gather(dstw, [splat_i(272)])[0]
  cur = curv[0]
  ends_after = nxt == cur
  is_first = cont & (first_done == 0)

  @pl.when(is_first)
  def _():
    for cc in range(8):
      pstage[pl.ds(16 * cc, 16)] = a0[cc]
      pstage[pl.ds(128 + 16 * cc, 16)] = a1[cc]
    mv = mstage[...]
    mv = jnp.where(lanes == 0, curv.astype(jnp.float32), mv)
    mv = jnp.where(lanes == 1, posv, mv)
    mstage[...] = mv

  @pl.when(jnp.logical_not(is_first) & ends_after)
  def _():
    for cc in range(8):
      pstage[pl.ds(256 + 16 * cc, 16)] = a0[cc]
      pstage[pl.ds(384 + 16 * cc, 16)] = a1[cc]
    mv = mstage[...]
    mv = jnp.where(lanes == 8, curv.astype(jnp.float32), mv)
    mv = jnp.where(lanes == 9, posv, mv)
    mstage[...] = mv

  @pl.when(jnp.logical_not(is_first) & jnp.logical_not(ends_after))
  def _():
    reset_outdst()
    for cc in range(8):
      out_t0[0, pl.ds(16 * cc, 16)] = a0[cc]
      out_t1[0, pl.ds(16 * cc, 16)] = a1[cc]
    dv0 = out_dst[pl.ds(0, 16)]
    out_dst[pl.ds(0, 16)] = jnp.where(lane0, curv, dv0)
    drain()

  pltpu.sync_copy(pstage, part_hbm.at[pl.ds(w * 512, 512)])
  pltpu.sync_copy(mstage, meta_hbm.at[pl.ds(w * 16, 16)])


@functools.partial(
    pl.kernel,
    out_type=(
        jax.ShapeDtypeStruct((NREC, X), jnp.float32),
        jax.ShapeDtypeStruct((NREC, X), jnp.float32),
        jax.ShapeDtypeStruct((NREC,), jnp.int32),
    ),
    mesh=_mesh,
    compiler_params=_sc_params,
    scratch_types=[
        pltpu.VMEM((NW * 512,), jnp.float32),
        pltpu.VMEM((NW * 16,), jnp.float32),
        pltpu.VMEM((NREC, X), jnp.float32),
        pltpu.VMEM((NREC, X), jnp.float32),
        pltpu.VMEM((NREC,), jnp.int32),
        pltpu.SemaphoreType.DMA,
    ],
)
def _merge(part_hbm, meta_hbm, cr_hbm, cl_hbm, bx_hbm,
           pv, mv, cr_v, cl_v, bx_v, sem):
  w = _wid()
  lanes = lax.iota(jnp.int32, 16)
  zero16 = jnp.zeros((16,), jnp.float32)

  pltpu.sync_copy(part_hbm, pv)
  pltpu.sync_copy(meta_hbm, mv)
  neg1 = jnp.full((16,), -1, jnp.int32)
  for r in range(NREC // 16):
    bx_v[pl.ds(16 * r, 16)] = neg1
  for r in range(NREC):
    for j8 in range(8):
      cr_v[r, pl.ds(16 * j8, 16)] = zero16
      cl_v[r, pl.ds(16 * j8, 16)] = zero16

  def rec(r, c):
    (have, cur, cnt, nfin) = c[:4]
    a0 = list(c[4:12])
    a1 = list(c[12:20])
    tile = r // 2
    slot = r - 2 * tile
    mb = tile * 16 + slot * 8
    pdstf = mv[pl.ds(mb, 16)][0]
    pcnt = mv[pl.ds(mb, 16)][1]
    pdst = pdstf.astype(jnp.int32)
    present = pdst >= 0
    same = present & (have == 1) & (pdst == cur)
    newopen = present & jnp.logical_not(same)
    fin_now = newopen & (have == 1)

    @pl.when(fin_now)
    def _():
      ridx = jnp.full((16,), nfin, jnp.int32)
      for j8 in range(8):
        cols = 16 * j8 + lanes
        plsc.store_scatter(cr_v, [ridx, cols], a0[j8])
        plsc.store_scatter(cl_v, [ridx, cols], a1[j8])
      plsc.store_scatter(bx_v, [ridx],
                         jnp.full((16,), cur, jnp.int32), mask=lanes == 0)

    nfin2 = nfin + fin_now.astype(jnp.int32)
    pb = tile * 512 + slot * 256
    na0, na1 = [], []
    for j8 in range(8):
      t0v = pv[pl.ds(pb + 16 * j8, 16)]
      t1v = pv[pl.ds(pb + 128 + 16 * j8, 16)]
      n0 = jnp.where(same, a0[j8] + t0v, jnp.where(newopen, t0v, a0[j8]))
      n1 = jnp.where(same, a1[j8] + t1v + cnt * t0v,
                     jnp.where(newopen, t1v, a1[j8]))
      na0.append(n0)
      na1.append(n1)
    cnt2 = jnp.where(same, cnt + pcnt, jnp.where(newopen, pcnt, cnt))
    cur2 = jnp.where(newopen, pdst, cur)
    have2 = jnp.where(newopen, 1, have)
    return tuple([have2, cur2, cnt2, nfin2] + na0 + na1)

  init = tuple([jnp.int32(0), jnp.int32(-1), jnp.float32(0.0), jnp.int32(0)]
               + [zero16] * 16)
  fin = lax.fori_loop(0, NREC, rec, init)
  (have, cur, cnt, nfin) = fin[:4]

  @pl.when(have == 1)
  def _():
    fa0, fa1 = list(fin[4:12]), list(fin[12:20])
    ridx = jnp.full((16,), nfin, jnp.int32)
    for j8 in range(8):
      cols = 16 * j8 + lanes
      plsc.store_scatter(cr_v, [ridx, cols], fa0[j8])
      plsc.store_scatter(cl_v, [ridx, cols], fa1[j8])
    plsc.store_scatter(bx_v, [ridx],
                       jnp.full((16,), cur, jnp.int32), mask=lanes == 0)

  @pl.when(w == 0)
  def _():
    pltpu.sync_copy(cr_v, cr_hbm)
    pltpu.sync_copy(cl_v, cl_hbm)
    pltpu.sync_copy(bx_v, bx_hbm)


BM = 512


def _conv_body(t0, t1, h0, deg, bidx, c0, c1, wr, wl, wt, b, h_out):
  i = pl.program_id(0)
  rel = bidx[...] - i * BM
  sel = (lax.broadcasted_iota(jnp.int32, (BM, NREC), 0) == rel)
  self_f = sel.astype(jnp.float32)
  hit = jnp.sum(self_f, axis=1, keepdims=True)
  t0_e = jnp.where(hit > 0.0,
                   jnp.dot(self_f, c0[...], preferred_element_type=jnp.float32),
                   t0[...])
  t1_e = jnp.where(hit > 0.0,
                   jnp.dot(self_f, c1[...], preferred_element_type=jnp.float32),
                   t1[...])
  degv = deg[...]
  inv = 1.0 / jnp.maximum(degv - 1.0, 1.0)
  sr_e = jnp.where(degv == 1.0, 0.5 * t0_e, t1_e * inv)
  sl_e = t0_e - sr_e
  cs = jnp.dot(sr_e, wr[...], preferred_element_type=jnp.float32)
  cs += jnp.dot(sl_e, wl[...], preferred_element_type=jnp.float32)
  cs += jnp.dot(h0[...], wt[...], preferred_element_type=jnp.float32)
  hn = jnp.maximum(cs + b[...], 0.0)
  h_out[...] = jnp.where(deg[...] > 0.0, hn, h0[...])


def _conv(sr, sl, h0, deg, bidx, cr, cl, wr, wl, wt, b):
  nb = N_PAD // BM
  blk = lambda i: (i, 0)
  cst = lambda i: (0, 0)
  return pl.pallas_call(
      _conv_body,
      grid=(nb,),
      in_specs=[
          pl.BlockSpec((BM, X), blk),
          pl.BlockSpec((BM, X), blk),
          pl.BlockSpec((BM, X), blk),
          pl.BlockSpec((BM, 1), blk),
          pl.BlockSpec((1, NREC), cst),
          pl.BlockSpec((NREC, X), cst),
          pl.BlockSpec((NREC, X), cst),
          pl.BlockSpec((X, H), cst),
          pl.BlockSpec((X, H), cst),
          pl.BlockSpec((X, H), cst),
          pl.BlockSpec((1, H), cst),
      ],
      out_specs=pl.BlockSpec((BM, H), blk),
      out_shape=jax.ShapeDtypeStruct((N_PAD, H), jnp.float32),
  )(sr, sl, h0, deg, bidx, cr, cl, wr, wl, wt, b)


NEG = -1e30


def _pool_body(h, gid, gw, gb, cw, cb, out, m_s, z_s, p_s):
  i = pl.program_id(0)
  nb = pl.num_programs(0)

  @pl.when(i == 0)
  def _():
    m_s[...] = jnp.full((G, 1), NEG, jnp.float32)
    z_s[...] = jnp.zeros((G, 1), jnp.float32)
    p_s[...] = jnp.zeros((G, H), jnp.float32)

  hb = h[...]
  s = lax.dot_general(gw[...], hb, (((1,), (1,)), ((), ()))) + gb[0, 0]
  gids = gid[0]
  mask = lax.broadcasted_iota(jnp.int32, (G, BM), 0) == gids
  bmax = jnp.max(jnp.where(mask, s, NEG), axis=1, keepdims=True)
  m_old = m_s[...]
  m_new = jnp.maximum(m_old, bmax)
  corr = jnp.where(m_new == m_old, 1.0, jnp.exp(m_old - m_new))
  m_s[...] = m_new
  e = jnp.where(mask, jnp.exp(s - m_new), 0.0)
  z_s[...] = z_s[...] * corr + jnp.sum(e, axis=1, keepdims=True)
  p_s[...] = p_s[...] * corr + jnp.dot(e, hb, preferred_element_type=jnp.float32)

  @pl.when(i == nb - 1)
  def _():
    pooled = p_s[...] / jnp.maximum(z_s[...], 1e-30)
    out[...] = lax.dot_general(
        pooled, cw[...], (((1,), (1,)), ((), ()))) + cb[...]


def _pool(h, gid, gw, gb, cw, cb):
  nb = N_PAD // BM
  blk = lambda i: (i, 0)
  cst = lambda i: (0, 0)
  return pl.pallas_call(
      _pool_body,
      grid=(nb,),
      in_specs=[
          pl.BlockSpec((BM, H), blk),
          pl.BlockSpec((1, 1, BM), lambda i: (i, 0, 0)),
          pl.BlockSpec((1, H), cst),
          pl.BlockSpec((1, 1), cst),
          pl.BlockSpec((NCLS, H), cst),
          pl.BlockSpec((1, NCLS), cst),
      ],
      out_specs=pl.BlockSpec((G, NCLS), cst),
      out_shape=jax.ShapeDtypeStruct((G, NCLS), jnp.float32),
      scratch_shapes=[
          pltpu.VMEM((G, 1), jnp.float32),
          pltpu.VMEM((G, 1), jnp.float32),
          pltpu.VMEM((G, H), jnp.float32),
      ],
  )(h, gid, gw, gb, cw, cb)


def kernel(node_types, edge_index, graph_ids, emb, W_left, W_right, W_top,
           b_conv, gate_W, gate_b, cls_W, cls_b):
  nt = node_types.astype(jnp.int32)
  src = edge_index[0].astype(jnp.int32)
  dst = edge_index[1].astype(jnp.int32)
  gid = graph_ids.astype(jnp.int32)

  nt_pad = jnp.concatenate([nt, jnp.zeros((N_PAD - N,), jnp.int32)])
  src_arr = jnp.concatenate([
      jnp.zeros((16,), jnp.int32), src, jnp.zeros((LA - 16 - E,), jnp.int32)])
  dst_arr = jnp.concatenate([
      jnp.full((16,), -1, jnp.int32), dst, jnp.full((LA - 16 - E,), N,
                                                    jnp.int32)])
  gid_pad = jnp.concatenate([gid, jnp.full((N_PAD - N,), G, jnp.int32)])

  deg = jax.ops.segment_sum(jnp.ones((E,), jnp.float32), dst, num_segments=N,
                            indices_are_sorted=True)
  deg_pad = jnp.concatenate([deg, jnp.zeros((N_PAD - N,), jnp.float32)])

  h0 = _emb_gather(nt_pad, emb)
  s_r, s_l, part, meta = _edge_pass(src_arr, dst_arr, h0)
  c_r, c_l, bidx = _merge(part, meta)
  h = _conv(s_r, s_l, h0, deg_pad.reshape(N_PAD, 1), bidx.reshape(1, NREC),
            c_r, c_l, W_right, W_left, W_top, b_conv)
  logits = _pool(h, gid_pad.reshape(N_PAD // BM, 1, BM), gate_W,
                 gate_b.reshape(1, 1), cls_W, cls_b.reshape(1, NCLS))
  return logits


# speedup vs baseline: 1.0880x; 1.0125x over previous
import functools

import jax
import jax.numpy as jnp
from jax import lax
from jax.experimental import pallas as pl
from jax.experimental.pallas import tpu as pltpu
from jax.experimental.pallas import tpu_sc as plsc

N = 100000
X = 128
H = 128
G = 256
NCLS = 104

NC = 2
NS = 16
NW = NC * NS

N_PAD = 102400
E = N - 1
C_EDGE = 3328
NWIN = C_EDGE // 256
K_WIN = 256
E_PAD = NW * C_EDGE
LA = 16 + E_PAD + 16
OUTB = 256
NREC = 2 * NW

_mesh = plsc.VectorSubcoreMesh(
    core_axis_name="c", subcore_axis_name="s", num_cores=NC, num_subcores=NS)
_sc_params = pltpu.CompilerParams(needs_layout_passes=False)


def _wid():
  return lax.axis_index("s") * NC + lax.axis_index("c")


EB = 320
EN = N_PAD // NW // EB


@functools.partial(
    pl.kernel,
    out_type=jax.ShapeDtypeStruct((N_PAD, X), jnp.float32),
    mesh=_mesh,
    compiler_params=_sc_params,
    scratch_types=[
        pltpu.VMEM((EB,), jnp.int32),
        pltpu.VMEM((EB,), jnp.int32),
        pltpu.VMEM((EB, X), jnp.float32),
        pltpu.VMEM((EB, X), jnp.float32),
        pltpu.SemaphoreType.DMA,
        pltpu.SemaphoreType.DMA,
        pltpu.SemaphoreType.DMA,
        pltpu.SemaphoreType.DMA,
        pltpu.SemaphoreType.DMA,
        pltpu.SemaphoreType.DMA,
    ],
)
def _emb_gather(nt_hbm, emb_hbm, h0_hbm, idx_a, idx_b, rows_a, rows_b,
                si_a, si_b, sg_a, sg_b, so_a, so_b):
  w = _wid()
  base0 = w * (N_PAD // NW)
  idx = [idx_a, idx_b]
  rows = [rows_a, rows_b]
  si = [si_a, si_b]
  sg = [sg_a, sg_b]
  so = [so_a, so_b]

  def bb(i):
    return pl.multiple_of(base0 + i * EB, 64)

  pltpu.async_copy(nt_hbm.at[pl.ds(bb(0), EB)], idx[0], si[0])
  si_d = [None, None]
  for i in range(EN):
    p = i & 1
    q = p ^ 1
    if i + 1 < EN:
      pltpu.async_copy(nt_hbm.at[pl.ds(bb(i + 1), EB)], idx[q], si[q])
    if i >= 2:
      pltpu.make_async_copy(rows[p], h0_hbm.at[pl.ds(bb(i - 2), EB)],
                            so[p]).wait()
    pltpu.make_async_copy(nt_hbm.at[pl.ds(bb(i), EB)], idx[p], si[p]).wait()
    pltpu.async_copy(emb_hbm.at[idx[p]], rows[p], sg[p]).wait()
    pltpu.async_copy(rows[p], h0_hbm.at[pl.ds(bb(i), EB)], so[p])
  for i in (EN - 2, EN - 1):
    p = i & 1
    pltpu.make_async_copy(rows[p], h0_hbm.at[pl.ds(bb(i), EB)], so[p]).wait()


@functools.partial(
    pl.kernel,
    out_type=(
        jax.ShapeDtypeStruct((N_PAD, X), jnp.float32),
        jax.ShapeDtypeStruct((N_PAD, X), jnp.float32),
        jax.ShapeDtypeStruct((NW * 512,), jnp.float32),
        jax.ShapeDtypeStruct((NW * 16,), jnp.float32),
    ),
    mesh=_mesh,
    compiler_params=_sc_params,
    scratch_types=[
        pltpu.VMEM((288,), jnp.int32),
        pltpu.VMEM((K_WIN,), jnp.int32),
        pltpu.VMEM((K_WIN, X), jnp.float32),
        pltpu.VMEM((OUTB, X), jnp.float32),
        pltpu.VMEM((OUTB, X), jnp.float32),
        pltpu.VMEM((OUTB,), jnp.int32),
        pltpu.VMEM((OUTB,), jnp.float32),
        pltpu.VMEM((512,), jnp.float32),
        pltpu.VMEM((16,), jnp.float32),
        pltpu.SemaphoreType.DMA,
    ],
)
def _edge_pass(src_hbm, dst_hbm, h0_hbm, t0_hbm, t1_hbm, part_hbm, meta_hbm,
               dstw, srcw, rows, out_t0, out_t1, out_dst, out_cnt,
               pstage, mstage, sem):
  w = _wid()
  t0 = w * C_EDGE
  dump = N + 16
  lanes = lax.iota(jnp.int32, 16)
  lane0 = lanes == 0
  zero16 = jnp.zeros((16,), jnp.float32)
  cols = [16 * c + lanes for c in range(8)]

  def reset_outdst():
    for j in range(OUTB // 16):
      out_dst[pl.ds(16 * j, 16)] = dump + 16 * j + lanes

  def drain():
    pltpu.async_copy(out_t0, t0_hbm.at[out_dst], sem).wait()
    pltpu.async_copy(out_t1, t1_hbm.at[out_dst], sem).wait()

  def splat_i(x):
    return jnp.full((16,), x, jnp.int32)

  mstage[...] = jnp.where((lanes == 0) | (lanes == 8), -1.0, 0.0)

  def stage(i):
    base = pl.multiple_of(t0 + i * K_WIN, 8)
    pltpu.sync_copy(dst_hbm.at[pl.ds(base, 288)], dstw)
    pltpu.sync_copy(src_hbm.at[pl.ds(base + 16, K_WIN)], srcw)
    pltpu.async_copy(h0_hbm.at[srcw], rows, sem).wait()

  stage(0)
  curv = plsc.load_gather(dstw, [splat_i(15)])
  cur0 = curv[0]
  cont = plsc.load_gather(dstw, [splat_i(16)])[0] == cur0
  first_done = jnp.where(cont, jnp.int32(0), jnp.int32(1))
  posv = zero16
  a0 = [zero16] * 8
  a1 = [zero16] * 8

  def window(i, wc):
    curv, posv, first_done = wc[0], wc[1], wc[2]
    a0 = list(wc[3:11])
    a1 = list(wc[11:19])

    @pl.when(i > 0)
    def _():
      stage(i)

    reset_outdst()

    def edge(j, c):
      dvidx, curv, posv, noutv = c[:4]
      a0 = list(c[4:12])
      a1 = list(c[12:20])
      dv = plsc.load_gather(dstw, [dvidx])
      changed = dv != curv
      flush = changed & (posv > 0.0)
      flushd = flush & lane0
      for cc in range(8):
        plsc.store_scatter(out_t0, [noutv, cols[cc]], a0[cc], mask=flush)
        plsc.store_scatter(out_t1, [noutv, cols[cc]], a1[cc], mask=flush)
      plsc.store_scatter(out_dst, [noutv], curv, mask=flushd)
      plsc.store_scatter(out_cnt, [noutv], posv, mask=flushd)
      noutv2 = noutv + jnp.where(flush, 1, 0).astype(jnp.int32)
      posb = jnp.where(changed, 0.0, posv)
      jv = dvidx - 16
      na0, na1 = [], []
      for cc in range(8):
        r = plsc.load_gather(rows, [jv, cols[cc]])
        na0.append(jnp.where(changed, zero16, a0[cc]) + r)
        na1.append(jnp.where(changed, zero16, a1[cc]) + posb * r)
      return tuple([dvidx + 1, dv, posb + 1.0, noutv2] + na0 + na1)

    init = tuple([splat_i(16), curv, posv, splat_i(0)] + a0 + a1)
    fin = lax.fori_loop(0, K_WIN, edge, init)
    curv, posv = fin[1], fin[2]
    noutv = fin[3]
    a0 = list(fin[4:12])
    a1 = list(fin[12:20])

    nflushed = noutv[0]
    do_fix = cont & (first_done == 0) & (nflushed > 0)

    @pl.when(do_fix)
    def _():
      for cc in range(8):
        pstage[pl.ds(16 * cc, 16)] = out_t0[0, pl.ds(16 * cc, 16)]
        pstage[pl.ds(128 + 16 * cc, 16)] = out_t1[0, pl.ds(16 * cc, 16)]
      cnt0 = plsc.load_gather(out_cnt, [splat_i(0)])
      mv = mstage[...]
      mv = jnp.where(lanes == 0, cur0.astype(jnp.float32), mv)
      mv = jnp.where(lanes == 1, cnt0, mv)
      mstage[...] = mv
      dv0 = out_dst[pl.ds(0, 16)]
      out_dst[pl.ds(0, 16)] = jnp.where(lane0, dump, dv0)

    first_done = jnp.where(do_fix, jnp.int32(1), first_done)
    drain()
    return tuple([curv, posv, first_done] + a0 + a1)

  init_w = tuple([curv, posv, first_done] + a0 + a1)
  fw = lax.fori_loop(0, NWIN, window, init_w)
  curv, posv, first_done = fw[0], fw[1], fw[2]
  a0 = list(fw[3:11])
  a1 = list(fw[11:19])

  nxt = plsc.load_gather(dstw, [splat_i(272)])[0]
  cur = curv[0]
  ends_after = nxt == cur
  is_first = cont & (first_done == 0)

  @pl.when(is_first)
  def _():
    for cc in range(8):
      pstage[pl.ds(16 * cc, 16)] = a0[cc]
      pstage[pl.ds(128 + 16 * cc, 16)] = a1[cc]
    mv = mstage[...]
    mv = jnp.where(lanes == 0, curv.astype(jnp.float32), mv)
    mv = jnp.where(lanes == 1, posv, mv)
    mstage[...] = mv

  @pl.when(jnp.logical_not(is_first) & ends_after)
  def _():
    for cc in range(8):
      pstage[pl.ds(256 + 16 * cc, 16)] = a0[cc]
      pstage[pl.ds(384 + 16 * cc, 16)] = a1[cc]
    mv = mstage[...]
    mv = jnp.where(lanes == 8, curv.astype(jnp.float32), mv)
    mv = jnp.where(lanes == 9, posv, mv)
    mstage[...] = mv

  @pl.when(jnp.logical_not(is_first) & jnp.logical_not(ends_after))
  def _():
    reset_outdst()
    for cc in range(8):
      out_t0[0, pl.ds(16 * cc, 16)] = a0[cc]
      out_t1[0, pl.ds(16 * cc, 16)] = a1[cc]
    dv0 = out_dst[pl.ds(0, 16)]
    out_dst[pl.ds(0, 16)] = jnp.where(lane0, curv, dv0)
    drain()

  pltpu.sync_copy(pstage, part_hbm.at[pl.ds(w * 512, 512)])
  pltpu.sync_copy(mstage, meta_hbm.at[pl.ds(w * 16, 16)])


@functools.partial(
    pl.kernel,
    out_type=(
        jax.ShapeDtypeStruct((NREC, X), jnp.float32),
        jax.ShapeDtypeStruct((NREC, X), jnp.float32),
        jax.ShapeDtypeStruct((NREC,), jnp.int32),
    ),
    mesh=_mesh,
    compiler_params=_sc_params,
    scratch_types=[
        pltpu.VMEM((NW * 512,), jnp.float32),
        pltpu.VMEM((NW * 16,), jnp.float32),
        pltpu.VMEM((NREC, X), jnp.float32),
        pltpu.VMEM((NREC, X), jnp.float32),
        pltpu.VMEM((NREC,), jnp.int32),
        pltpu.SemaphoreType.DMA,
    ],
)
def _merge(part_hbm, meta_hbm, cr_hbm, cl_hbm, bx_hbm,
           pv, mv, cr_v, cl_v, bx_v, sem):
  w = _wid()
  lanes = lax.iota(jnp.int32, 16)
  zero16 = jnp.zeros((16,), jnp.float32)

  pltpu.sync_copy(part_hbm, pv)
  pltpu.sync_copy(meta_hbm, mv)
  neg1 = jnp.full((16,), -1, jnp.int32)
  for r in range(NREC // 16):
    bx_v[pl.ds(16 * r, 16)] = neg1
  for r in range(NREC):
    for j8 in range(8):
      cr_v[r, pl.ds(16 * j8, 16)] = zero16
      cl_v[r, pl.ds(16 * j8, 16)] = zero16

  def rec(r, c):
    (have, cur, cnt, nfin) = c[:4]
    a0 = list(c[4:12])
    a1 = list(c[12:20])
    tile = r // 2
    slot = r - 2 * tile
    mb = tile * 16 + slot * 8
    pdstf = mv[pl.ds(mb, 16)][0]
    pcnt = mv[pl.ds(mb, 16)][1]
    pdst = pdstf.astype(jnp.int32)
    present = pdst >= 0
    same = present & (have == 1) & (pdst == cur)
    newopen = present & jnp.logical_not(same)
    fin_now = newopen & (have == 1)

    @pl.when(fin_now)
    def _():
      ridx = jnp.full((16,), nfin, jnp.int32)
      for j8 in range(8):
        cols = 16 * j8 + lanes
        plsc.store_scatter(cr_v, [ridx, cols], a0[j8])
        plsc.store_scatter(cl_v, [ridx, cols], a1[j8])
      plsc.store_scatter(bx_v, [ridx],
                         jnp.full((16,), cur, jnp.int32), mask=lanes == 0)

    nfin2 = nfin + fin_now.astype(jnp.int32)
    pb = tile * 512 + slot * 256
    na0, na1 = [], []
    for j8 in range(8):
      t0v = pv[pl.ds(pb + 16 * j8, 16)]
      t1v = pv[pl.ds(pb + 128 + 16 * j8, 16)]
      n0 = jnp.where(same, a0[j8] + t0v, jnp.where(newopen, t0v, a0[j8]))
      n1 = jnp.where(same, a1[j8] + t1v + cnt * t0v,
                     jnp.where(newopen, t1v, a1[j8]))
      na0.append(n0)
      na1.append(n1)
    cnt2 = jnp.where(same, cnt + pcnt, jnp.where(newopen, pcnt, cnt))
    cur2 = jnp.where(newopen, pdst, cur)
    have2 = jnp.where(newopen, 1, have)
    return tuple([have2, cur2, cnt2, nfin2] + na0 + na1)

  init = tuple([jnp.int32(0), jnp.int32(-1), jnp.float32(0.0), jnp.int32(0)]
               + [zero16] * 16)
  fin = lax.fori_loop(0, NREC, rec, init)
  (have, cur, cnt, nfin) = fin[:4]

  @pl.when(have == 1)
  def _():
    fa0, fa1 = list(fin[4:12]), list(fin[12:20])
    ridx = jnp.full((16,), nfin, jnp.int32)
    for j8 in range(8):
      cols = 16 * j8 + lanes
      plsc.store_scatter(cr_v, [ridx, cols], fa0[j8])
      plsc.store_scatter(cl_v, [ridx, cols], fa1[j8])
    plsc.store_scatter(bx_v, [ridx],
                       jnp.full((16,), cur, jnp.int32), mask=lanes == 0)

  @pl.when(w == 0)
  def _():
    pltpu.sync_copy(cr_v, cr_hbm)
    pltpu.sync_copy(cl_v, cl_hbm)
    pltpu.sync_copy(bx_v, bx_hbm)


BM = 512


def _conv_body(t0, t1, h0, deg, bidx, c0, c1, wr, wl, wt, b, h_out):
  i = pl.program_id(0)
  rel = bidx[...] - i * BM
  sel = (lax.broadcasted_iota(jnp.int32, (BM, NREC), 0) == rel)
  self_f = sel.astype(jnp.float32)
  hit = jnp.sum(self_f, axis=1, keepdims=True)
  t0_e = jnp.where(hit > 0.0,
                   jnp.dot(self_f, c0[...], preferred_element_type=jnp.float32),
                   t0[...])
  t1_e = jnp.where(hit > 0.0,
                   jnp.dot(self_f, c1[...], preferred_element_type=jnp.float32),
                   t1[...])
  degv = deg[...]
  inv = 1.0 / jnp.maximum(degv - 1.0, 1.0)
  sr_e = jnp.where(degv == 1.0, 0.5 * t0_e, t1_e * inv)
  sl_e = t0_e - sr_e
  cs = jnp.dot(sr_e, wr[...], preferred_element_type=jnp.float32)
  cs += jnp.dot(sl_e, wl[...], preferred_element_type=jnp.float32)
  cs += jnp.dot(h0[...], wt[...], preferred_element_type=jnp.float32)
  hn = jnp.maximum(cs + b[...], 0.0)
  h_out[...] = jnp.where(deg[...] > 0.0, hn, h0[...])


def _conv(sr, sl, h0, deg, bidx, cr, cl, wr, wl, wt, b):
  nb = N_PAD // BM
  blk = lambda i: (i, 0)
  cst = lambda i: (0, 0)
  return pl.pallas_call(
      _conv_body,
      grid=(nb,),
      in_specs=[
          pl.BlockSpec((BM, X), blk),
          pl.BlockSpec((BM, X), blk),
          pl.BlockSpec((BM, X), blk),
          pl.BlockSpec((BM, 1), blk),
          pl.BlockSpec((1, NREC), cst),
          pl.BlockSpec((NREC, X), cst),
          pl.BlockSpec((NREC, X), cst),
          pl.BlockSpec((X, H), cst),
          pl.BlockSpec((X, H), cst),
          pl.BlockSpec((X, H), cst),
          pl.BlockSpec((1, H), cst),
      ],
      out_specs=pl.BlockSpec((BM, H), blk),
      out_shape=jax.ShapeDtypeStruct((N_PAD, H), jnp.float32),
  )(sr, sl, h0, deg, bidx, cr, cl, wr, wl, wt, b)


NEG = -1e30


def _pool_body(h, gid, gw, gb, cw, cb, out, m_s, z_s, p_s):
  i = pl.program_id(0)
  nb = pl.num_programs(0)

  @pl.when(i == 0)
  def _():
    m_s[...] = jnp.full((G, 1), NEG, jnp.float32)
    z_s[...] = jnp.zeros((G, 1), jnp.float32)
    p_s[...] = jnp.zeros((G, H), jnp.float32)

  hb = h[...]
  s = lax.dot_general(gw[...], hb, (((1,), (1,)), ((), ()))) + gb[0, 0]
  gids = gid[0]
  mask = lax.broadcasted_iota(jnp.int32, (G, BM), 0) == gids
  bmax = jnp.max(jnp.where(mask, s, NEG), axis=1, keepdims=True)
  m_old = m_s[...]
  m_new = jnp.maximum(m_old, bmax)
  corr = jnp.where(m_new == m_old, 1.0, jnp.exp(m_old - m_new))
  m_s[...] = m_new
  e = jnp.where(mask, jnp.exp(s - m_new), 0.0)
  z_s[...] = z_s[...] * corr + jnp.sum(e, axis=1, keepdims=True)
  p_s[...] = p_s[...] * corr + jnp.dot(e, hb, preferred_element_type=jnp.float32)

  @pl.when(i == nb - 1)
  def _():
    pooled = p_s[...] / jnp.maximum(z_s[...], 1e-30)
    out[...] = lax.dot_general(
        pooled, cw[...], (((1,), (1,)), ((), ()))) + cb[...]


def _pool(h, gid, gw, gb, cw, cb):
  nb = N_PAD // BM
  blk = lambda i: (i, 0)
  cst = lambda i: (0, 0)
  return pl.pallas_call(
      _pool_body,
      grid=(nb,),
      in_specs=[
          pl.BlockSpec((BM, H), blk),
          pl.BlockSpec((1, 1, BM), lambda i: (i, 0, 0)),
          pl.BlockSpec((1, H), cst),
          pl.BlockSpec((1, 1), cst),
          pl.BlockSpec((NCLS, H), cst),
          pl.BlockSpec((1, NCLS), cst),
      ],
      out_specs=pl.BlockSpec((G, NCLS), cst),
      out_shape=jax.ShapeDtypeStruct((G, NCLS), jnp.float32),
      scratch_shapes=[
          pltpu.VMEM((G, 1), jnp.float32),
          pltpu.VMEM((G, 1), jnp.float32),
          pltpu.VMEM((G, H), jnp.float32),
      ],
  )(h, gid, gw, gb, cw, cb)


def kernel(node_types, edge_index, graph_ids, emb, W_left, W_right, W_top,
           b_conv, gate_W, gate_b, cls_W, cls_b):
  nt = node_types.astype(jnp.int32)
  src = edge_index[0].astype(jnp.int32)
  dst = edge_index[1].astype(jnp.int32)
  gid = graph_ids.astype(jnp.int32)

  nt_pad = jnp.concatenate([nt, jnp.zeros((N_PAD - N,), jnp.int32)])
  src_arr = jnp.concatenate([
      jnp.zeros((16,), jnp.int32), src, jnp.zeros((LA - 16 - E,), jnp.int32)])
  dst_arr = jnp.concatenate([
      jnp.full((16,), -1, jnp.int32), dst, jnp.full((LA - 16 - E,), N,
                                                    jnp.int32)])
  gid_pad = jnp.concatenate([gid, jnp.full((N_PAD - N,), G, jnp.int32)])

  deg = jax.ops.segment_sum(jnp.ones((E,), jnp.float32), dst, num_segments=N,
                            indices_are_sorted=True)
  deg_pad = jnp.concatenate([deg, jnp.zeros((N_PAD - N,), jnp.float32)])

  h0 = _emb_gather(nt_pad, emb)
  s_r, s_l, part, meta = _edge_pass(src_arr, dst_arr, h0)
  c_r, c_l, bidx = _merge(part, meta)
  h = _conv(s_r, s_l, h0, deg_pad.reshape(N_PAD, 1), bidx.reshape(1, NREC),
            c_r, c_l, W_right, W_left, W_top, b_conv)
  logits = _pool(h, gid_pad.reshape(N_PAD // BM, 1, BM), gate_W,
                 gate_b.reshape(1, 1), cls_W, cls_b.reshape(1, NCLS))
  return logits
